# baseline (speedup 1.0000x reference)
import functools

import jax
import jax.numpy as jnp
from jax import lax
from jax.experimental import pallas as pl
from jax.experimental.pallas import tpu as pltpu
from jax.experimental.pallas import tpu_sc as plsc

_B = 4096
_S = 200
_D = 64
_NW = 32
_TOTAL = _B * _S

_GRP = 100
_NGRP = 16
_CHUNK = _GRP * _NGRP
_ROWS_PER_W = _TOTAL // _NW
_NCHUNK = _ROWS_PER_W // _CHUNK
_BR_PER_CHUNK = _CHUNK // _S


def _emb_kernel(idx_hbm, tok_hbm, pos_hbm, out_hbm, idx_v, rows_v, pos_v, sem):
    wid = lax.axis_index("s") * 2 + lax.axis_index("c")

    pltpu.sync_copy(pos_hbm.at[pl.ds(0, _S)], pos_v)

    def chunk_body(c, carry):
        base = pl.multiple_of(wid * _ROWS_PER_W + c * _CHUNK, _CHUNK)

        irow = pl.multiple_of(base // _GRP, _NGRP)
        pltpu.sync_copy(idx_hbm.at[pl.ds(irow, _NGRP)], idx_v)

        copies = []
        for j in range(_NGRP):
            copies.append(
                pltpu.async_copy(
                    tok_hbm.at[idx_v.at[j]],
                    rows_v.at[pl.ds(j * _GRP, _GRP)],
                    sem,
                )
            )
        for cp in copies:
            cp.wait()

        def add_body(s, carry2):
            for br in range(_BR_PER_CHUNK):
                r = br * _S
                for half in range(2):
                    t = rows_v[r + s, pl.ds(half * 32, 32)]
                    p = pos_v[s, pl.ds(half * 32, 32)]
                    rows_v[r + s, pl.ds(half * 32, 32)] = t + p
            return carry2

        lax.fori_loop(0, _S, add_body, 0)

        pltpu.sync_copy(rows_v, out_hbm.at[pl.ds(base, _CHUNK)])
        return carry

    lax.fori_loop(0, _NCHUNK, chunk_body, 0)


@jax.jit
def kernel(x, token_table, pos_table):
    idx = x.reshape(_TOTAL // _GRP, _GRP).astype(jnp.int32)

    mesh = plsc.VectorSubcoreMesh(core_axis_name="c", subcore_axis_name="s")
    out = pl.kernel(
        _emb_kernel,
        mesh=mesh,
        compiler_params=pltpu.CompilerParams(use_tc_tiling_on_sc=False),
        out_type=jax.ShapeDtypeStruct((_TOTAL, _D), jnp.bfloat16),
        scratch_types=[
            pltpu.VMEM((_NGRP, _GRP), jnp.int32),
            pltpu.VMEM((_CHUNK, _D), jnp.bfloat16),
            pltpu.VMEM((_S, _D), jnp.bfloat16),
            pltpu.SemaphoreType.DMA,
        ],
    )(idx, token_table, pos_table)

    return out.reshape(_B, _S, _D)
